# baseline (speedup 1.0000x reference)
import functools

import jax
import jax.numpy as jnp
from jax import lax
from jax.experimental import pallas as pl
from jax.experimental.pallas import tpu as pltpu
from jax.experimental.pallas import tpu_sc as plsc

_N = 100000
_CIN = 16
_COUT = 16
_KVOL = 27
_NW = 32
_BW = 3200
_NP = _NW * _BW
_ZROW = _N
_BN = 3200
_BC = 400
_NCH = _BW // _BC
_GW = 512
_NZ = 128
_NF = _N + _NZ


_NH = 4
_NPH = _NP // _NH
_BWH = _NPH // _NW
_NCHH = _BWH // _BC
_NKCH = _KVOL * _NCHH


def _make_sc_gather(h):
    def body(feats_hbm, kmap_hbm, mask_hbm, g_hbm,
             kv0, kv1, mv0, mv1, iv0, iv1, buf0, buf1, fsh,
             skm0, skm1, sg0, sg1, sw0, sw1):
        kvs, mvs, ivs, bufs = (kv0, kv1), (mv0, mv1), (iv0, iv1), (buf0, buf1)
        skms, sgs, sws = (skm0, skm1), (sg0, sg1), (sw0, sw1)
        sid = lax.axis_index("s")
        wid = sid * 2 + lax.axis_index("c")
        base = wid * _BWH
        zv = _ZROW + lax.iota(jnp.int32, 16) * 8 + (wid % 8)

        @pl.when(sid == 0)
        def _():
            pltpu.sync_copy(feats_hbm, fsh)

        def z_body(r, c):
            buf0[r, :] = jnp.zeros((_CIN,), jnp.float32)
            return c

        lax.fori_loop(0, _BC, z_body, 0)

        def zc_body(zc, c):
            cb = base + (zc // 5) * _BC
            col = _KVOL * _CIN + (zc % 5) * _CIN
            pltpu.sync_copy(buf0, g_hbm.at[pl.ds(cb, _BC), pl.ds(col, _CIN)])
            return c

        lax.fori_loop(0, _NCHH * 5, zc_body, 0)

        plsc.subcore_barrier()

        def rowbase(kc):
            return (kc // _NCHH) * _NP + h * _NPH + base + (kc % _NCHH) * _BC

        def gslice(kc):
            cb = base + (kc % _NCHH) * _BC
            return g_hbm.at[pl.ds(cb, _BC), pl.ds((kc // _NCHH) * _CIN, _CIN)]

        def issue_km(kc, b):
            pltpu.async_copy(kmap_hbm.at[pl.ds(rowbase(kc), _BC)], kvs[b], skms[b])
            pltpu.async_copy(mask_hbm.at[pl.ds(rowbase(kc), _BC)], mvs[b], skms[b])

        def wait_km(b):
            pltpu.make_async_copy(kmap_hbm.at[pl.ds(0, _BC)], kvs[b], skms[b]).wait()
            pltpu.make_async_copy(mask_hbm.at[pl.ds(0, _BC)], mvs[b], skms[b]).wait()

        issue_km(0, 0)

        def loop_body(i2, c):
            for b in (0, 1):
                kc = i2 * 2 + b
                nb = 1 - b

                @pl.when(kc >= 1)
                def _():
                    pltpu.make_async_copy(fsh.at[ivs[nb]], bufs[nb], sgs[nb]).wait()
                    pltpu.async_copy(bufs[nb], gslice(kc - 1), sws[nb])

                @pl.when(kc + 1 < _NKCH)
                def _():
                    issue_km(kc + 1, nb)

                wait_km(b)

                def g_body(g, cc):
                    s = pl.ds(g * 16, 16)
                    ivs[b][s] = jnp.where(mvs[b][s] != 0, kvs[b][s], zv)
                    return cc

                lax.fori_loop(0, _BC // 16, g_body, 0)

                @pl.when(kc >= 2)
                def _():
                    pltpu.make_async_copy(bufs[b], gslice(kc - 2), sws[b]).wait()

                pltpu.async_copy(fsh.at[ivs[b]], bufs[b], sgs[b])
            return c

        lax.fori_loop(0, _NKCH // 2, loop_body, 0)

        pltpu.make_async_copy(fsh.at[ivs[1]], bufs[1], sgs[1]).wait()
        pltpu.async_copy(bufs[1], gslice(_NKCH - 1), sws[1])
        pltpu.make_async_copy(bufs[0], gslice(_NKCH - 2), sws[0]).wait()
        pltpu.make_async_copy(bufs[1], gslice(_NKCH - 1), sws[1]).wait()

    return functools.partial(
        pl.kernel,
        out_type=jax.ShapeDtypeStruct((_NPH, _GW), jnp.float32),
        mesh=plsc.VectorSubcoreMesh(core_axis_name="c", subcore_axis_name="s"),
        scratch_types=[
            pltpu.VMEM((_BC,), jnp.int32),
            pltpu.VMEM((_BC,), jnp.int32),
            pltpu.VMEM((_BC,), jnp.int32),
            pltpu.VMEM((_BC,), jnp.int32),
            pltpu.VMEM((_BC,), jnp.int32),
            pltpu.VMEM((_BC,), jnp.int32),
            pltpu.VMEM((_BC, _CIN), jnp.float32),
            pltpu.VMEM((_BC, _CIN), jnp.float32),
            pltpu.VMEM_SHARED((_NF, _CIN), jnp.float32),
            pltpu.SemaphoreType.DMA,
            pltpu.SemaphoreType.DMA,
            pltpu.SemaphoreType.DMA,
            pltpu.SemaphoreType.DMA,
            pltpu.SemaphoreType.DMA,
            pltpu.SemaphoreType.DMA,
        ],
        compiler_params=pltpu.CompilerParams(use_tc_tiling_on_sc=False),
    )(body)


_sc_gathers = [_make_sc_gather(h) for h in range(_NH)]


def _tc_reduce_body(g_ref, w_ref, b_ref, o_ref):
    o_ref[...] = jnp.broadcast_to(b_ref[...], (_BN, _COUT)) + jnp.dot(
        g_ref[...], w_ref[...], preferred_element_type=jnp.float32)


_tc_reduce = pl.pallas_call(
    _tc_reduce_body,
    grid=(_NPH // _BN,),
    in_specs=[
        pl.BlockSpec((_BN, _GW), lambda n: (n, 0)),
        pl.BlockSpec((_GW, _COUT), lambda n: (0, 0)),
        pl.BlockSpec((1, _COUT), lambda n: (0, 0)),
    ],
    out_specs=pl.BlockSpec((_BN, _COUT), lambda n: (n, 0)),
    out_shape=jax.ShapeDtypeStruct((_NPH, _COUT), jnp.float32),
)


def kernel(feats, kmap, mask, weight, bias):
    feats_pad = jnp.zeros((_NF, _CIN), jnp.float32).at[:_N].set(feats)
    kmap1 = jnp.pad(kmap.astype(jnp.int32), ((0, 0), (0, _NP - _N))).reshape(-1)
    mask1 = jnp.pad(mask.astype(jnp.int32), ((0, 0), (0, _NP - _N))).reshape(-1)
    wflat = jnp.pad(weight.reshape(_KVOL * _CIN, _COUT),
                    ((0, _GW - _KVOL * _CIN), (0, 0)))
    bias2 = bias.reshape(1, _COUT)
    outs = [_tc_reduce(_sc_gathers[h](feats_pad, kmap1, mask1), wflat, bias2)
            for h in range(_NH)]
    return jnp.concatenate(outs, axis=0)[:_N]


# speedup vs baseline: 1.0510x; 1.0510x over previous
import functools

import jax
import jax.numpy as jnp
from jax import lax
from jax.experimental import pallas as pl
from jax.experimental.pallas import tpu as pltpu
from jax.experimental.pallas import tpu_sc as plsc

_N = 100000
_CIN = 16
_COUT = 16
_KVOL = 27
_NW = 32
_BW = 3200
_NP = _NW * _BW
_ZROW = _N
_BN = 2048
_BC = 800
_NCH = _BW // _BC
_GW = 512
_NZ = 128
_NF = _N + _NZ


_NH = 2
_NPH = _NP // _NH
_BWH = _NPH // _NW
_NCHH = _BWH // _BC
_NKCH = _KVOL * _NCHH


def _make_sc_gather(h):
    def body(feats_hbm, kmap_hbm, mask_hbm, g_hbm,
             kv0, kv1, mv0, mv1, iv0, iv1, buf0, buf1, fsh,
             skm0, skm1, sg0, sg1, sw0, sw1):
        kvs, mvs, ivs, bufs = (kv0, kv1), (mv0, mv1), (iv0, iv1), (buf0, buf1)
        skms, sgs, sws = (skm0, skm1), (sg0, sg1), (sw0, sw1)
        sid = lax.axis_index("s")
        wid = sid * 2 + lax.axis_index("c")
        base = wid * _BWH
        zv = _ZROW + lax.iota(jnp.int32, 16) * 8 + (wid % 8)

        @pl.when(sid == 0)
        def _():
            pltpu.sync_copy(feats_hbm, fsh)

        plsc.subcore_barrier()

        def rowbase(kc):
            return (kc // _NCHH) * _NP + h * _NPH + base + (kc % _NCHH) * _BC

        def gslice(kc):
            cb = base + (kc % _NCHH) * _BC
            return g_hbm.at[pl.ds(cb, _BC), pl.ds((kc // _NCHH) * _CIN, _CIN)]

        def issue_km(kc, b):
            pltpu.async_copy(kmap_hbm.at[pl.ds(rowbase(kc), _BC)], kvs[b], skms[b])
            pltpu.async_copy(mask_hbm.at[pl.ds(rowbase(kc), _BC)], mvs[b], skms[b])

        def wait_km(b):
            pltpu.make_async_copy(kmap_hbm.at[pl.ds(0, _BC)], kvs[b], skms[b]).wait()
            pltpu.make_async_copy(mask_hbm.at[pl.ds(0, _BC)], mvs[b], skms[b]).wait()

        issue_km(0, 0)

        def loop_body(i2, c):
            for b in (0, 1):
                kc = i2 * 2 + b
                nb = 1 - b

                @pl.when(kc >= 1)
                def _():
                    pltpu.make_async_copy(fsh.at[ivs[nb]], bufs[nb], sgs[nb]).wait()
                    pltpu.async_copy(bufs[nb], gslice(kc - 1), sws[nb])

                @pl.when(kc + 1 < _NKCH)
                def _():
                    issue_km(kc + 1, nb)

                wait_km(b)

                def g_body(g, cc):
                    s = pl.ds(g * 16, 16)
                    ivs[b][s] = jnp.where(mvs[b][s] != 0, kvs[b][s], zv)
                    return cc

                lax.fori_loop(0, _BC // 16, g_body, 0)

                @pl.when(kc >= 2)
                def _():
                    pltpu.make_async_copy(bufs[b], gslice(kc - 2), sws[b]).wait()

                pltpu.async_copy(fsh.at[ivs[b]], bufs[b], sgs[b])
            return c

        lax.fori_loop(0, _NKCH // 2, loop_body, 0)

        pltpu.make_async_copy(fsh.at[ivs[1]], bufs[1], sgs[1]).wait()
        pltpu.async_copy(bufs[1], gslice(_NKCH - 1), sws[1])
        pltpu.make_async_copy(bufs[0], gslice(_NKCH - 2), sws[0]).wait()
        pltpu.make_async_copy(bufs[1], gslice(_NKCH - 1), sws[1]).wait()

    return functools.partial(
        pl.kernel,
        out_type=jax.ShapeDtypeStruct((_NPH, _GW), jnp.float32),
        mesh=plsc.VectorSubcoreMesh(core_axis_name="c", subcore_axis_name="s"),
        scratch_types=[
            pltpu.VMEM((_BC,), jnp.int32),
            pltpu.VMEM((_BC,), jnp.int32),
            pltpu.VMEM((_BC,), jnp.int32),
            pltpu.VMEM((_BC,), jnp.int32),
            pltpu.VMEM((_BC,), jnp.int32),
            pltpu.VMEM((_BC,), jnp.int32),
            pltpu.VMEM((_BC, _CIN), jnp.float32),
            pltpu.VMEM((_BC, _CIN), jnp.float32),
            pltpu.VMEM_SHARED((_NF, _CIN), jnp.float32),
            pltpu.SemaphoreType.DMA,
            pltpu.SemaphoreType.DMA,
            pltpu.SemaphoreType.DMA,
            pltpu.SemaphoreType.DMA,
            pltpu.SemaphoreType.DMA,
            pltpu.SemaphoreType.DMA,
        ],
        compiler_params=pltpu.CompilerParams(use_tc_tiling_on_sc=False),
    )(body)


_sc_gathers = [_make_sc_gather(h) for h in range(_NH)]


_GU = _KVOL * _CIN


def _tc_reduce_body(g_ref, w_ref, b_ref, o_ref):
    o_ref[...] = jnp.broadcast_to(b_ref[...], (_BN, _COUT)) + jnp.dot(
        g_ref[:, :_GU], w_ref[...], preferred_element_type=jnp.float32)


_tc_reduce = pl.pallas_call(
    _tc_reduce_body,
    grid=(_NPH // _BN,),
    in_specs=[
        pl.BlockSpec((_BN, _GW), lambda n: (n, 0)),
        pl.BlockSpec((_GU, _COUT), lambda n: (0, 0)),
        pl.BlockSpec((1, _COUT), lambda n: (0, 0)),
    ],
    out_specs=pl.BlockSpec((_BN, _COUT), lambda n: (n, 0)),
    out_shape=jax.ShapeDtypeStruct((_NPH, _COUT), jnp.float32),
)


def kernel(feats, kmap, mask, weight, bias):
    feats_pad = jnp.zeros((_NF, _CIN), jnp.float32).at[:_N].set(feats)
    kmap1 = jnp.pad(kmap.astype(jnp.int32), ((0, 0), (0, _NP - _N))).reshape(-1)
    mask1 = jnp.pad(mask.astype(jnp.int32), ((0, 0), (0, _NP - _N))).reshape(-1)
    wflat = weight.reshape(_KVOL * _CIN, _COUT)
    bias2 = bias.reshape(1, _COUT)
    outs = [_tc_reduce(_sc_gathers[h](feats_pad, kmap1, mask1), wflat, bias2)
            for h in range(_NH)]
    return jnp.concatenate(outs, axis=0)[:_N]


# speedup vs baseline: 1.0666x; 1.0148x over previous
import functools

import jax
import jax.numpy as jnp
from jax import lax
from jax.experimental import pallas as pl
from jax.experimental.pallas import tpu as pltpu
from jax.experimental.pallas import tpu_sc as plsc

_N = 100000
_CIN = 16
_COUT = 16
_KVOL = 27
_NW = 32
_BW = 3200
_NP = _NW * _BW
_ZROW = _N
_BN = 6400
_BC = 800
_NCH = _BW // _BC
_GW = 512
_NZ = 128
_NF = _N + _NZ


_NH = 2
_NPH = _NP // _NH
_BWH = _NPH // _NW
_NCHH = _BWH // _BC
_NKCH = _KVOL * _NCHH


def _make_sc_gather(h):
    def body(feats_hbm, kmap_hbm, mask_hbm, g_hbm,
             kv0, kv1, mv0, mv1, iv0, iv1, buf0, buf1, fsh,
             skm0, skm1, sg0, sg1, sw0, sw1):
        kvs, mvs, ivs, bufs = (kv0, kv1), (mv0, mv1), (iv0, iv1), (buf0, buf1)
        skms, sgs, sws = (skm0, skm1), (sg0, sg1), (sw0, sw1)
        sid = lax.axis_index("s")
        wid = sid * 2 + lax.axis_index("c")
        base = wid * _BWH
        zv = _ZROW + lax.iota(jnp.int32, 16) * 8 + (wid % 8)

        @pl.when(sid == 0)
        def _():
            pltpu.sync_copy(feats_hbm, fsh)

        plsc.subcore_barrier()

        def rowbase(kc):
            return (kc // _NCHH) * _NP + h * _NPH + base + (kc % _NCHH) * _BC

        def gslice(kc):
            cb = base + (kc % _NCHH) * _BC
            return g_hbm.at[pl.ds(cb, _BC), pl.ds((kc // _NCHH) * _CIN, _CIN)]

        def issue_km(kc, b):
            pltpu.async_copy(kmap_hbm.at[pl.ds(rowbase(kc), _BC)], kvs[b], skms[b])
            pltpu.async_copy(mask_hbm.at[pl.ds(rowbase(kc), _BC)], mvs[b], skms[b])

        def wait_km(b):
            pltpu.make_async_copy(kmap_hbm.at[pl.ds(0, _BC)], kvs[b], skms[b]).wait()
            pltpu.make_async_copy(mask_hbm.at[pl.ds(0, _BC)], mvs[b], skms[b]).wait()

        issue_km(0, 0)

        def loop_body(i2, c):
            for b in (0, 1):
                kc = i2 * 2 + b
                nb = 1 - b

                @pl.when(kc >= 1)
                def _():
                    pltpu.make_async_copy(fsh.at[ivs[nb]], bufs[nb], sgs[nb]).wait()
                    pltpu.async_copy(bufs[nb], gslice(kc - 1), sws[nb])

                @pl.when(kc + 1 < _NKCH)
                def _():
                    issue_km(kc + 1, nb)

                wait_km(b)

                def g_body(g, cc):
                    s = pl.ds(g * 16, 16)
                    ivs[b][s] = jnp.where(mvs[b][s] != 0, kvs[b][s], zv)
                    return cc

                lax.fori_loop(0, _BC // 16, g_body, 0)

                @pl.when(kc >= 2)
                def _():
                    pltpu.make_async_copy(bufs[b], gslice(kc - 2), sws[b]).wait()

                pltpu.async_copy(fsh.at[ivs[b]], bufs[b], sgs[b])
            return c

        lax.fori_loop(0, _NKCH // 2, loop_body, 0)

        pltpu.make_async_copy(fsh.at[ivs[1]], bufs[1], sgs[1]).wait()
        pltpu.async_copy(bufs[1], gslice(_NKCH - 1), sws[1])
        pltpu.make_async_copy(bufs[0], gslice(_NKCH - 2), sws[0]).wait()
        pltpu.make_async_copy(bufs[1], gslice(_NKCH - 1), sws[1]).wait()

    return functools.partial(
        pl.kernel,
        out_type=jax.ShapeDtypeStruct((_NPH, _GW), jnp.float32),
        mesh=plsc.VectorSubcoreMesh(core_axis_name="c", subcore_axis_name="s"),
        scratch_types=[
            pltpu.VMEM((_BC,), jnp.int32),
            pltpu.VMEM((_BC,), jnp.int32),
            pltpu.VMEM((_BC,), jnp.int32),
            pltpu.VMEM((_BC,), jnp.int32),
            pltpu.VMEM((_BC,), jnp.int32),
            pltpu.VMEM((_BC,), jnp.int32),
            pltpu.VMEM((_BC, _CIN), jnp.float32),
            pltpu.VMEM((_BC, _CIN), jnp.float32),
            pltpu.VMEM_SHARED((_NF, _CIN), jnp.float32),
            pltpu.SemaphoreType.DMA,
            pltpu.SemaphoreType.DMA,
            pltpu.SemaphoreType.DMA,
            pltpu.SemaphoreType.DMA,
            pltpu.SemaphoreType.DMA,
            pltpu.SemaphoreType.DMA,
        ],
        compiler_params=pltpu.CompilerParams(use_tc_tiling_on_sc=False),
    )(body)


_sc_gathers = [_make_sc_gather(h) for h in range(_NH)]


_GU = _KVOL * _CIN


def _tc_reduce_body(g_ref, w_ref, b_ref, o_ref):
    o_ref[...] = jnp.broadcast_to(b_ref[...], (_BN, _COUT)) + jnp.dot(
        g_ref[:, :_GU], w_ref[...], preferred_element_type=jnp.float32)


_tc_reduce = pl.pallas_call(
    _tc_reduce_body,
    grid=(_NPH // _BN,),
    in_specs=[
        pl.BlockSpec((_BN, _GW), lambda n: (n, 0)),
        pl.BlockSpec((_GU, _COUT), lambda n: (0, 0)),
        pl.BlockSpec((1, _COUT), lambda n: (0, 0)),
    ],
    out_specs=pl.BlockSpec((_BN, _COUT), lambda n: (n, 0)),
    out_shape=jax.ShapeDtypeStruct((_NPH, _COUT), jnp.float32),
)


def kernel(feats, kmap, mask, weight, bias):
    feats_pad = jnp.zeros((_NF, _CIN), jnp.float32).at[:_N].set(feats)
    kmap1 = jnp.pad(kmap.astype(jnp.int32), ((0, 0), (0, _NP - _N))).reshape(-1)
    mask1 = jnp.pad(mask.astype(jnp.int32), ((0, 0), (0, _NP - _N))).reshape(-1)
    wflat = weight.reshape(_KVOL * _CIN, _COUT)
    bias2 = bias.reshape(1, _COUT)
    outs = [_tc_reduce(_sc_gathers[h](feats_pad, kmap1, mask1), wflat, bias2)
            for h in range(_NH)]
    return jnp.concatenate(outs, axis=0)[:_N]
